# Initial kernel scaffold; baseline (speedup 1.0000x reference)
import jax
import jax.numpy as jnp
from jax.experimental import pallas as pl


def kernel(x, pos, edge_attr, edge_index, batch_idx, emb_W, emb_b, msg_W1, msg_b1, msg_W2, msg_b2, upd_W1, upd_b1, upd_W2, upd_b2, pred_W1, pred_b1, pred_W2, pred_b2):
    raise NotImplementedError("write your pallas kernel here")


# speedup vs baseline: 1.9832x; 1.9832x over previous
import functools

import jax
import jax.numpy as jnp
from jax import lax
from jax.experimental import pallas as pl
from jax.experimental.pallas import tpu as pltpu
from jax.experimental.pallas import tpu_sc as plsc

DEPTH = 4
HID = 128
NF = 128
NR = 4
OUT = 1
N = 10000
E = 320000
NG = 64

F32 = jnp.float32

NC = 2
NS = 16
NW = NC * NS
EPW = E // NW
CH = 80
NCHUNK = EPW // CH
NPAD = 10240
RPT = NPAD // NS

BN = 2000
BE = 2000

def _mesh():
    return plsc.VectorSubcoreMesh(
        core_axis_name="c", subcore_axis_name="s", num_cores=NC, num_subcores=NS
    )


def _silu(x):
    return x * lax.logistic(x)


def _ln(x):
    m = jnp.mean(x, axis=-1, keepdims=True)
    v = jnp.mean((x - m) * (x - m), axis=-1, keepdims=True)
    return (x - m) * lax.rsqrt(v + 1e-5)


_SC_CACHE = {}


def _sc_gather_body(a_hbm, b_hbm, src_hbm, dst_hbm, ga_hbm, gb_hbm,
                    src_v, dst_v, ar_v, br_v, sem_a, sem_b):
    c = lax.axis_index("c")
    s = lax.axis_index("s")
    base = (c * NS + s) * EPW

    def body(i, carry):
        off = base + i * CH
        pltpu.sync_copy(src_hbm.at[pl.ds(off, CH)], src_v)
        pltpu.sync_copy(dst_hbm.at[pl.ds(off, CH)], dst_v)
        da = pltpu.async_copy(a_hbm.at[src_v], ar_v, sem_a)
        db = pltpu.async_copy(b_hbm.at[dst_v], br_v, sem_b)
        da.wait()
        db.wait()
        pltpu.sync_copy(ar_v, ga_hbm.at[pl.ds(off, CH)])
        pltpu.sync_copy(br_v, gb_hbm.at[pl.ds(off, CH)])
        return carry

    lax.fori_loop(0, NCHUNK, body, 0)


def _sc_gather(a, b, src, dst):
    k = _SC_CACHE.get("gather")
    if k is None:
        k = pl.kernel(
            _sc_gather_body,
            out_type=(
                jax.ShapeDtypeStruct((E, HID), F32),
                jax.ShapeDtypeStruct((E, HID), F32),
            ),
            mesh=_mesh(),
            scratch_types=[
                pltpu.VMEM((CH,), jnp.int32),
                pltpu.VMEM((CH,), jnp.int32),
                pltpu.VMEM((CH, HID), F32),
                pltpu.VMEM((CH, HID), F32),
                pltpu.SemaphoreType.DMA,
                pltpu.SemaphoreType.DMA,
            ],
        )
        _SC_CACHE["gather"] = k
    return k(a, b, src, dst)


def _sc_pos_gather_body(p_hbm, src_hbm, dst_hbm, ps_hbm, pd_hbm,
                        src_v, dst_v, pr_v, qr_v, sem_a, sem_b):
    c = lax.axis_index("c")
    s = lax.axis_index("s")
    base = (c * NS + s) * EPW

    def body(i, carry):
        off = base + i * CH
        pltpu.sync_copy(src_hbm.at[pl.ds(off, CH)], src_v)
        pltpu.sync_copy(dst_hbm.at[pl.ds(off, CH)], dst_v)
        da = pltpu.async_copy(p_hbm.at[src_v], pr_v, sem_a)
        db = pltpu.async_copy(p_hbm.at[dst_v], qr_v, sem_b)
        da.wait()
        db.wait()
        pltpu.sync_copy(pr_v, ps_hbm.at[pl.ds(off, CH)])
        pltpu.sync_copy(qr_v, pd_hbm.at[pl.ds(off, CH)])
        return carry

    lax.fori_loop(0, NCHUNK, body, 0)


def _sc_pos_gather(pos16, src, dst):
    k = _SC_CACHE.get("pos")
    if k is None:
        k = pl.kernel(
            _sc_pos_gather_body,
            out_type=(
                jax.ShapeDtypeStruct((E, 16), F32),
                jax.ShapeDtypeStruct((E, 16), F32),
            ),
            mesh=_mesh(),
            scratch_types=[
                pltpu.VMEM((CH,), jnp.int32),
                pltpu.VMEM((CH,), jnp.int32),
                pltpu.VMEM((CH, 16), F32),
                pltpu.VMEM((CH, 16), F32),
                pltpu.SemaphoreType.DMA,
                pltpu.SemaphoreType.DMA,
            ],
            compiler_params=pltpu.CompilerParams(use_tc_tiling_on_sc=False),
        )
        _SC_CACHE["pos"] = k
    return k(pos16, src, dst)


def _sc_scatter_body(m2_hbm, dst_hbm, zeros_hbm, out_hbm, dst_v, rows_v, acc_sh):
    c = lax.axis_index("c")
    s = lax.axis_index("s")
    base = (c * NS + s) * EPW

    pltpu.sync_copy(zeros_hbm.at[pl.ds(s * RPT, RPT)],
                    acc_sh.at[pl.ds(s * RPT, RPT)])
    plsc.subcore_barrier()

    def body(i, carry):
        off = base + i * CH
        pltpu.sync_copy(dst_hbm.at[pl.ds(off, CH)], dst_v)
        pltpu.sync_copy(m2_hbm.at[pl.ds(off, CH)], rows_v)
        pltpu.sync_copy(rows_v, acc_sh.at[dst_v], add=True)
        return carry

    lax.fori_loop(0, NCHUNK, body, 0)
    plsc.subcore_barrier()
    pltpu.sync_copy(acc_sh.at[pl.ds(s * RPT, RPT)],
                    out_hbm.at[c, pl.ds(s * RPT, RPT)])


def _sc_scatter(m2, dst, zeros_n):
    k = _SC_CACHE.get("scatter")
    if k is None:
        k = pl.kernel(
            _sc_scatter_body,
            out_type=jax.ShapeDtypeStruct((NC, NPAD, HID), F32),
            mesh=_mesh(),
            scratch_types=[
                pltpu.VMEM((CH,), jnp.int32),
                pltpu.VMEM((CH, HID), F32),
                pltpu.VMEM_SHARED((NPAD, HID), F32),
            ],
        )
        _SC_CACHE["scatter"] = k
    return k(m2, dst, zeros_n)


def _emb_body(x_ref, w_ref, b_ref, o_ref):
    o_ref[...] = (
        jnp.dot(x_ref[...], w_ref[...], preferred_element_type=F32, precision=lax.Precision.HIGHEST) + b_ref[...]
    )


def _emb_call(x, w, b):
    return pl.pallas_call(
        _emb_body,
        grid=(N // BN,),
        in_specs=[
            pl.BlockSpec((BN, NF), lambda i: (i, 0)),
            pl.BlockSpec((NF, HID), lambda i: (0, 0)),
            pl.BlockSpec((1, HID), lambda i: (0, 0)),
        ],
        out_specs=pl.BlockSpec((BN, HID), lambda i: (i, 0)),
        out_shape=jax.ShapeDtypeStruct((N, HID), F32),
    )(x, w, b)


def _proj_body(h_ref, w_ref, a_ref, b_ref):
    h = h_ref[...]
    w = w_ref[...]
    a_ref[...] = jnp.dot(h, w[:HID], preferred_element_type=F32, precision=lax.Precision.HIGHEST)
    b_ref[...] = jnp.dot(h, w[HID:2 * HID], preferred_element_type=F32, precision=lax.Precision.HIGHEST)


def _proj_call(h, w1):
    return pl.pallas_call(
        _proj_body,
        grid=(N // BN,),
        in_specs=[
            pl.BlockSpec((BN, HID), lambda i: (i, 0)),
            pl.BlockSpec((2 * HID, HID), lambda i: (0, 0)),
        ],
        out_specs=[
            pl.BlockSpec((BN, HID), lambda i: (i, 0)),
            pl.BlockSpec((BN, HID), lambda i: (i, 0)),
        ],
        out_shape=[
            jax.ShapeDtypeStruct((N, HID), F32),
            jax.ShapeDtypeStruct((N, HID), F32),
        ],
    )(h, w1)


def _msg_body(ga_ref, gb_ref, ps_ref, pd_ref, at_ref, wr_ref, w2_ref, b2_ref,
              o_ref):
    d = ps_ref[...] - pd_ref[...]
    d2 = jnp.sum(d * d, axis=-1, keepdims=True)
    a = at_ref[...]
    mx = jnp.max(a, axis=-1, keepdims=True)
    eq = a >= mx
    e0 = eq[:, 0:1]
    e1 = eq[:, 1:2] & ~e0
    e2 = eq[:, 2:3] & ~(e0 | e1)
    e3 = eq[:, 3:4] & ~(e0 | e1 | e2)
    wr = wr_ref[...]
    pre = (
        ga_ref[...] + gb_ref[...]
        + d2 * wr[0:1]
        + e0.astype(F32) * wr[1:2]
        + e1.astype(F32) * wr[2:3]
        + e2.astype(F32) * wr[3:4]
        + e3.astype(F32) * wr[4:5]
        + wr[5:6]
    )
    m = _silu(_ln(pre))
    y = jnp.dot(m, w2_ref[...], preferred_element_type=F32, precision=lax.Precision.HIGHEST) + b2_ref[...]
    o_ref[...] = _silu(y)


def _msg_call(ga, gb, ps, pd, attr, wrest, w2, b2):
    return pl.pallas_call(
        _msg_body,
        grid=(E // BE,),
        in_specs=[
            pl.BlockSpec((BE, HID), lambda i: (i, 0)),
            pl.BlockSpec((BE, HID), lambda i: (i, 0)),
            pl.BlockSpec((BE, 16), lambda i: (i, 0)),
            pl.BlockSpec((BE, 16), lambda i: (i, 0)),
            pl.BlockSpec((BE, NR), lambda i: (i, 0)),
            pl.BlockSpec((6, HID), lambda i: (0, 0)),
            pl.BlockSpec((HID, HID), lambda i: (0, 0)),
            pl.BlockSpec((1, HID), lambda i: (0, 0)),
        ],
        out_specs=pl.BlockSpec((BE, HID), lambda i: (i, 0)),
        out_shape=jax.ShapeDtypeStruct((E, HID), F32),
    )(ga, gb, ps, pd, attr, wrest, w2, b2)


def _upd_body(h_ref, p_ref, u1_ref, ub1_ref, u2_ref, ub2_ref, o_ref):
    h = h_ref[...]
    agg = p_ref[0] + p_ref[1]
    u1 = u1_ref[...]
    pre = (
        jnp.dot(h, u1[:HID], preferred_element_type=F32, precision=lax.Precision.HIGHEST)
        + jnp.dot(agg, u1[HID:], preferred_element_type=F32, precision=lax.Precision.HIGHEST)
        + ub1_ref[...]
    )
    u = _silu(_ln(pre))
    o_ref[...] = h + jnp.dot(u, u2_ref[...], preferred_element_type=F32, precision=lax.Precision.HIGHEST) + ub2_ref[...]


def _upd_call(h, partials, u1, ub1, u2, ub2):
    return pl.pallas_call(
        _upd_body,
        grid=(N // BN,),
        in_specs=[
            pl.BlockSpec((BN, HID), lambda i: (i, 0)),
            pl.BlockSpec((NC, BN, HID), lambda i: (0, i, 0)),
            pl.BlockSpec((2 * HID, HID), lambda i: (0, 0)),
            pl.BlockSpec((1, HID), lambda i: (0, 0)),
            pl.BlockSpec((HID, HID), lambda i: (0, 0)),
            pl.BlockSpec((1, HID), lambda i: (0, 0)),
        ],
        out_specs=pl.BlockSpec((BN, HID), lambda i: (i, 0)),
        out_shape=jax.ShapeDtypeStruct((N, HID), F32),
    )(h, partials, u1, ub1, u2, ub2)


def _pool_body(h_ref, bi_ref, w1_ref, b1_ref, w2t_ref, b2_ref, o_ref, acc):
    i = pl.program_id(0)

    @pl.when(i == 0)
    def _():
        acc[...] = jnp.zeros((NG, HID), F32)

    b = bi_ref[0]
    g = lax.broadcasted_iota(jnp.int32, (NG, BN), 0)
    oht = (g == b).astype(F32)
    acc[...] += jnp.dot(oht, h_ref[...], preferred_element_type=F32, precision=lax.Precision.HIGHEST)

    @pl.when(i == N // BN - 1)
    def _():
        z = jnp.dot(acc[...], w1_ref[...], preferred_element_type=F32, precision=lax.Precision.HIGHEST) + b1_ref[...]
        z = jnp.maximum(z, 0.0)
        o_ref[...] = jnp.sum(z * w2t_ref[...], axis=-1, keepdims=True) + b2_ref[...]


def _pool_call(h, bidx3, w1, b1, w2t, b2):
    return pl.pallas_call(
        _pool_body,
        grid=(N // BN,),
        in_specs=[
            pl.BlockSpec((BN, HID), lambda i: (i, 0)),
            pl.BlockSpec((1, 1, BN), lambda i: (i, 0, 0)),
            pl.BlockSpec((HID, HID), lambda i: (0, 0)),
            pl.BlockSpec((1, HID), lambda i: (0, 0)),
            pl.BlockSpec((1, HID), lambda i: (0, 0)),
            pl.BlockSpec((1, 1), lambda i: (0, 0)),
        ],
        out_specs=pl.BlockSpec((NG, OUT), lambda i: (0, 0)),
        out_shape=jax.ShapeDtypeStruct((NG, OUT), F32),
        scratch_shapes=[pltpu.VMEM((NG, HID), F32)],
    )(h, bidx3, w1, b1, w2t, b2)


def kernel(x, pos, edge_attr, edge_index, batch_idx, emb_W, emb_b,
           msg_W1, msg_b1, msg_W2, msg_b2, upd_W1, upd_b1, upd_W2, upd_b2,
           pred_W1, pred_b1, pred_W2, pred_b2):
    src = edge_index[0]
    dst = edge_index[1]
    pos16 = jnp.concatenate([pos, jnp.zeros((N, 13), F32)], axis=1)
    zeros_n = jnp.zeros((NPAD, HID), F32)
    bidx3 = batch_idx.reshape(N // BN, 1, BN)

    h = _emb_call(x, emb_W, emb_b.reshape(1, HID))
    ps, pd = _sc_pos_gather(pos16, src, dst)

    for l in range(DEPTH):
        a, b = _proj_call(h, msg_W1[l, : 2 * HID])
        ga, gb = _sc_gather(a, b, src, dst)
        wrest = jnp.concatenate(
            [msg_W1[l, 2 * HID:], msg_b1[l].reshape(1, HID)], axis=0
        )
        m2 = _msg_call(ga, gb, ps, pd, edge_attr, wrest,
                       msg_W2[l], msg_b2[l].reshape(1, HID))
        partials = _sc_scatter(m2, dst, zeros_n)[:, :N, :]
        h = _upd_call(h, partials, upd_W1[l], upd_b1[l].reshape(1, HID),
                      upd_W2[l], upd_b2[l].reshape(1, HID))

    return _pool_call(h, bidx3, pred_W1, pred_b1.reshape(1, HID),
                      pred_W2.reshape(1, HID), pred_b2.reshape(1, 1))


# speedup vs baseline: 2.5222x; 1.2718x over previous
import functools

import jax
import jax.numpy as jnp
from jax import lax
from jax.experimental import pallas as pl
from jax.experimental.pallas import tpu as pltpu
from jax.experimental.pallas import tpu_sc as plsc

DEPTH = 4
HID = 128
NF = 128
NR = 4
OUT = 1
N = 10000
E = 320000
NG = 64

F32 = jnp.float32

NC = 2
NS = 16
NW = NC * NS
EPW = E // NW
CH = 80
NCHUNK = EPW // CH
NPAD = 10240
RPT = NPAD // NS

BN = 2000
BE = 2000

def _mesh():
    return plsc.VectorSubcoreMesh(
        core_axis_name="c", subcore_axis_name="s", num_cores=NC, num_subcores=NS
    )


def _silu(x):
    return x * lax.logistic(x)


def _ln(x):
    m = jnp.mean(x, axis=-1, keepdims=True)
    v = jnp.mean((x - m) * (x - m), axis=-1, keepdims=True)
    return (x - m) * lax.rsqrt(v + 1e-5)


_SC_CACHE = {}
RB = 5
CHS = 40
NCHUNKS = EPW // CHS


def _sc_gather_body(a_hbm, b_hbm, src_hbm, dst_hbm, ga_hbm, gb_hbm, *scr):
    src_all, dst_all = scr[0], scr[1]
    abuf = scr[2:2 + RB]
    bbuf = scr[2 + RB:2 + 2 * RB]
    gsa = scr[2 + 2 * RB:2 + 3 * RB]
    gsb = scr[2 + 3 * RB:2 + 4 * RB]
    wsa = scr[2 + 4 * RB:2 + 5 * RB]
    wsb = scr[2 + 5 * RB:2 + 6 * RB]
    c = lax.axis_index("c")
    s = lax.axis_index("s")
    base = (c * NS + s) * EPW
    pltpu.sync_copy(src_hbm.at[pl.ds(base, EPW)], src_all)
    pltpu.sync_copy(dst_hbm.at[pl.ds(base, EPW)], dst_all)

    def outer(g, carry):
        for k in range(RB):
            i = g * RB + k
            loc = i * CH
            off = base + loc

            @pl.when(g > 0)
            def _():
                pltpu.make_async_copy(abuf[k], ga_hbm.at[pl.ds(off, CH)], wsa[k]).wait()
                pltpu.make_async_copy(bbuf[k], gb_hbm.at[pl.ds(off, CH)], wsb[k]).wait()

            pltpu.async_copy(a_hbm.at[src_all.at[pl.ds(loc, CH)]], abuf[k], gsa[k])
            pltpu.async_copy(b_hbm.at[dst_all.at[pl.ds(loc, CH)]], bbuf[k], gsb[k])
        for k in range(RB):
            i = g * RB + k
            loc = i * CH
            off = base + loc
            pltpu.make_async_copy(a_hbm.at[src_all.at[pl.ds(loc, CH)]], abuf[k], gsa[k]).wait()
            pltpu.make_async_copy(b_hbm.at[dst_all.at[pl.ds(loc, CH)]], bbuf[k], gsb[k]).wait()
            pltpu.async_copy(abuf[k], ga_hbm.at[pl.ds(off, CH)], wsa[k])
            pltpu.async_copy(bbuf[k], gb_hbm.at[pl.ds(off, CH)], wsb[k])
        return carry

    lax.fori_loop(0, NCHUNK // RB, outer, 0)
    for k in range(RB):
        pltpu.make_async_copy(abuf[k], ga_hbm.at[pl.ds(base, CH)], wsa[k]).wait()
        pltpu.make_async_copy(bbuf[k], gb_hbm.at[pl.ds(base, CH)], wsb[k]).wait()


def _sc_gather(a, b, src, dst):
    k = _SC_CACHE.get("gather")
    if k is None:
        scr = (
            [pltpu.VMEM((EPW,), jnp.int32)] * 2
            + [pltpu.VMEM((CH, HID), F32)] * (2 * RB)
            + [pltpu.SemaphoreType.DMA] * (4 * RB)
        )
        k = pl.kernel(
            _sc_gather_body,
            out_type=(
                jax.ShapeDtypeStruct((E, HID), F32),
                jax.ShapeDtypeStruct((E, HID), F32),
            ),
            mesh=_mesh(),
            scratch_types=scr,
        )
        _SC_CACHE["gather"] = k
    return k(a, b, src, dst)


def _sc_pos_gather_body(p_hbm, src_hbm, dst_hbm, ps_hbm, pd_hbm, *scr):
    src_all, dst_all = scr[0], scr[1]
    abuf = scr[2:2 + RB]
    bbuf = scr[2 + RB:2 + 2 * RB]
    gsa = scr[2 + 2 * RB:2 + 3 * RB]
    gsb = scr[2 + 3 * RB:2 + 4 * RB]
    wsa = scr[2 + 4 * RB:2 + 5 * RB]
    wsb = scr[2 + 5 * RB:2 + 6 * RB]
    c = lax.axis_index("c")
    s = lax.axis_index("s")
    base = (c * NS + s) * EPW
    pltpu.sync_copy(src_hbm.at[pl.ds(base, EPW)], src_all)
    pltpu.sync_copy(dst_hbm.at[pl.ds(base, EPW)], dst_all)

    def outer(g, carry):
        for k in range(RB):
            i = g * RB + k
            loc = i * CH
            off = base + loc

            @pl.when(g > 0)
            def _():
                pltpu.make_async_copy(abuf[k], ps_hbm.at[pl.ds(off, CH)], wsa[k]).wait()
                pltpu.make_async_copy(bbuf[k], pd_hbm.at[pl.ds(off, CH)], wsb[k]).wait()

            pltpu.async_copy(p_hbm.at[src_all.at[pl.ds(loc, CH)]], abuf[k], gsa[k])
            pltpu.async_copy(p_hbm.at[dst_all.at[pl.ds(loc, CH)]], bbuf[k], gsb[k])
        for k in range(RB):
            i = g * RB + k
            loc = i * CH
            off = base + loc
            pltpu.make_async_copy(p_hbm.at[src_all.at[pl.ds(loc, CH)]], abuf[k], gsa[k]).wait()
            pltpu.make_async_copy(p_hbm.at[dst_all.at[pl.ds(loc, CH)]], bbuf[k], gsb[k]).wait()
            pltpu.async_copy(abuf[k], ps_hbm.at[pl.ds(off, CH)], wsa[k])
            pltpu.async_copy(bbuf[k], pd_hbm.at[pl.ds(off, CH)], wsb[k])
        return carry

    lax.fori_loop(0, NCHUNK // RB, outer, 0)
    for k in range(RB):
        pltpu.make_async_copy(abuf[k], ps_hbm.at[pl.ds(base, CH)], wsa[k]).wait()
        pltpu.make_async_copy(bbuf[k], pd_hbm.at[pl.ds(base, CH)], wsb[k]).wait()


def _sc_pos_gather(pos16, src, dst):
    k = _SC_CACHE.get("pos")
    if k is None:
        scr = (
            [pltpu.VMEM((EPW,), jnp.int32)] * 2
            + [pltpu.VMEM((CH, 16), F32)] * (2 * RB)
            + [pltpu.SemaphoreType.DMA] * (4 * RB)
        )
        k = pl.kernel(
            _sc_pos_gather_body,
            out_type=(
                jax.ShapeDtypeStruct((E, 16), F32),
                jax.ShapeDtypeStruct((E, 16), F32),
            ),
            mesh=_mesh(),
            scratch_types=scr,
            compiler_params=pltpu.CompilerParams(use_tc_tiling_on_sc=False),
        )
        _SC_CACHE["pos"] = k
    return k(pos16, src, dst)


def _sc_scatter_body(m2_hbm, dst_hbm, zeros_hbm, out_hbm, *scr):
    acc_sh = scr[0]
    dbuf = scr[1:1 + RB]
    mbuf = scr[1 + RB:1 + 2 * RB]
    lsi = scr[1 + 2 * RB:1 + 3 * RB]
    lsm = scr[1 + 3 * RB:1 + 4 * RB]
    ssem = scr[1 + 4 * RB:1 + 5 * RB]
    c = lax.axis_index("c")
    s = lax.axis_index("s")
    base = (c * NS + s) * EPW

    pltpu.sync_copy(zeros_hbm.at[pl.ds(s * RPT, RPT)],
                    acc_sh.at[pl.ds(s * RPT, RPT)])
    plsc.subcore_barrier()

    def outer(g, carry):
        for k in range(RB):
            i = g * RB + k
            off = base + i * CHS

            @pl.when(g > 0)
            def _():
                pltpu.make_async_copy(mbuf[k], acc_sh.at[dbuf[k]], ssem[k]).wait()

            pltpu.async_copy(dst_hbm.at[pl.ds(off, CHS)], dbuf[k], lsi[k])
            pltpu.async_copy(m2_hbm.at[pl.ds(off, CHS)], mbuf[k], lsm[k])
        for k in range(RB):
            i = g * RB + k
            off = base + i * CHS
            pltpu.make_async_copy(dst_hbm.at[pl.ds(off, CHS)], dbuf[k], lsi[k]).wait()
            pltpu.make_async_copy(m2_hbm.at[pl.ds(off, CHS)], mbuf[k], lsm[k]).wait()
            pltpu.async_copy(mbuf[k], acc_sh.at[dbuf[k]], ssem[k], add=True)
        return carry

    lax.fori_loop(0, NCHUNKS // RB, outer, 0)
    for k in range(RB):
        pltpu.make_async_copy(mbuf[k], acc_sh.at[dbuf[k]], ssem[k]).wait()
    plsc.subcore_barrier()
    pltpu.sync_copy(acc_sh.at[pl.ds(s * RPT, RPT)],
                    out_hbm.at[c, pl.ds(s * RPT, RPT)])


def _sc_scatter(m2, dst, zeros_n):
    k = _SC_CACHE.get("scatter")
    if k is None:
        scr = (
            [pltpu.VMEM_SHARED((NPAD, HID), F32)]
            + [pltpu.VMEM((CHS,), jnp.int32)] * RB
            + [pltpu.VMEM((CHS, HID), F32)] * RB
            + [pltpu.SemaphoreType.DMA] * (3 * RB)
        )
        k = pl.kernel(
            _sc_scatter_body,
            out_type=jax.ShapeDtypeStruct((NC, NPAD, HID), F32),
            mesh=_mesh(),
            scratch_types=scr,
        )
        _SC_CACHE["scatter"] = k
    return k(m2, dst, zeros_n)


def _emb_body(x_ref, w_ref, b_ref, o_ref):
    o_ref[...] = (
        jnp.dot(x_ref[...], w_ref[...], preferred_element_type=F32, precision=lax.Precision.HIGHEST) + b_ref[...]
    )


def _emb_call(x, w, b):
    return pl.pallas_call(
        _emb_body,
        grid=(N // BN,),
        in_specs=[
            pl.BlockSpec((BN, NF), lambda i: (i, 0)),
            pl.BlockSpec((NF, HID), lambda i: (0, 0)),
            pl.BlockSpec((1, HID), lambda i: (0, 0)),
        ],
        out_specs=pl.BlockSpec((BN, HID), lambda i: (i, 0)),
        out_shape=jax.ShapeDtypeStruct((N, HID), F32),
    )(x, w, b)


def _proj_body(h_ref, w_ref, a_ref, b_ref):
    h = h_ref[...]
    w = w_ref[...]
    a_ref[...] = jnp.dot(h, w[:HID], preferred_element_type=F32, precision=lax.Precision.HIGHEST)
    b_ref[...] = jnp.dot(h, w[HID:2 * HID], preferred_element_type=F32, precision=lax.Precision.HIGHEST)


def _proj_call(h, w1):
    return pl.pallas_call(
        _proj_body,
        grid=(N // BN,),
        in_specs=[
            pl.BlockSpec((BN, HID), lambda i: (i, 0)),
            pl.BlockSpec((2 * HID, HID), lambda i: (0, 0)),
        ],
        out_specs=[
            pl.BlockSpec((BN, HID), lambda i: (i, 0)),
            pl.BlockSpec((BN, HID), lambda i: (i, 0)),
        ],
        out_shape=[
            jax.ShapeDtypeStruct((N, HID), F32),
            jax.ShapeDtypeStruct((N, HID), F32),
        ],
    )(h, w1)


def _msg_body(ga_ref, gb_ref, ps_ref, pd_ref, at_ref, wr_ref, w2_ref, b2_ref,
              o_ref):
    d = ps_ref[...] - pd_ref[...]
    d2 = jnp.sum(d * d, axis=-1, keepdims=True)
    a = at_ref[...]
    mx = jnp.max(a, axis=-1, keepdims=True)
    eq = a >= mx
    e0 = eq[:, 0:1]
    e1 = eq[:, 1:2] & ~e0
    e2 = eq[:, 2:3] & ~(e0 | e1)
    e3 = eq[:, 3:4] & ~(e0 | e1 | e2)
    wr = wr_ref[...]
    pre = (
        ga_ref[...] + gb_ref[...]
        + d2 * wr[0:1]
        + e0.astype(F32) * wr[1:2]
        + e1.astype(F32) * wr[2:3]
        + e2.astype(F32) * wr[3:4]
        + e3.astype(F32) * wr[4:5]
        + wr[5:6]
    )
    m = _silu(_ln(pre))
    y = jnp.dot(m, w2_ref[...], preferred_element_type=F32, precision=lax.Precision.HIGHEST) + b2_ref[...]
    o_ref[...] = _silu(y)


def _msg_call(ga, gb, ps, pd, attr, wrest, w2, b2):
    return pl.pallas_call(
        _msg_body,
        grid=(E // BE,),
        in_specs=[
            pl.BlockSpec((BE, HID), lambda i: (i, 0)),
            pl.BlockSpec((BE, HID), lambda i: (i, 0)),
            pl.BlockSpec((BE, 16), lambda i: (i, 0)),
            pl.BlockSpec((BE, 16), lambda i: (i, 0)),
            pl.BlockSpec((BE, NR), lambda i: (i, 0)),
            pl.BlockSpec((6, HID), lambda i: (0, 0)),
            pl.BlockSpec((HID, HID), lambda i: (0, 0)),
            pl.BlockSpec((1, HID), lambda i: (0, 0)),
        ],
        out_specs=pl.BlockSpec((BE, HID), lambda i: (i, 0)),
        out_shape=jax.ShapeDtypeStruct((E, HID), F32),
    )(ga, gb, ps, pd, attr, wrest, w2, b2)


def _upd_body(h_ref, p_ref, u1_ref, ub1_ref, u2_ref, ub2_ref, o_ref):
    h = h_ref[...]
    agg = p_ref[0] + p_ref[1]
    u1 = u1_ref[...]
    pre = (
        jnp.dot(h, u1[:HID], preferred_element_type=F32, precision=lax.Precision.HIGHEST)
        + jnp.dot(agg, u1[HID:], preferred_element_type=F32, precision=lax.Precision.HIGHEST)
        + ub1_ref[...]
    )
    u = _silu(_ln(pre))
    o_ref[...] = h + jnp.dot(u, u2_ref[...], preferred_element_type=F32, precision=lax.Precision.HIGHEST) + ub2_ref[...]


def _upd_call(h, partials, u1, ub1, u2, ub2):
    return pl.pallas_call(
        _upd_body,
        grid=(N // BN,),
        in_specs=[
            pl.BlockSpec((BN, HID), lambda i: (i, 0)),
            pl.BlockSpec((NC, BN, HID), lambda i: (0, i, 0)),
            pl.BlockSpec((2 * HID, HID), lambda i: (0, 0)),
            pl.BlockSpec((1, HID), lambda i: (0, 0)),
            pl.BlockSpec((HID, HID), lambda i: (0, 0)),
            pl.BlockSpec((1, HID), lambda i: (0, 0)),
        ],
        out_specs=pl.BlockSpec((BN, HID), lambda i: (i, 0)),
        out_shape=jax.ShapeDtypeStruct((N, HID), F32),
    )(h, partials, u1, ub1, u2, ub2)


def _pool_body(h_ref, bi_ref, w1_ref, b1_ref, w2t_ref, b2_ref, o_ref, acc):
    i = pl.program_id(0)

    @pl.when(i == 0)
    def _():
        acc[...] = jnp.zeros((NG, HID), F32)

    b = bi_ref[0]
    g = lax.broadcasted_iota(jnp.int32, (NG, BN), 0)
    oht = (g == b).astype(F32)
    acc[...] += jnp.dot(oht, h_ref[...], preferred_element_type=F32, precision=lax.Precision.HIGHEST)

    @pl.when(i == N // BN - 1)
    def _():
        z = jnp.dot(acc[...], w1_ref[...], preferred_element_type=F32, precision=lax.Precision.HIGHEST) + b1_ref[...]
        z = jnp.maximum(z, 0.0)
        o_ref[...] = jnp.sum(z * w2t_ref[...], axis=-1, keepdims=True) + b2_ref[...]


def _pool_call(h, bidx3, w1, b1, w2t, b2):
    return pl.pallas_call(
        _pool_body,
        grid=(N // BN,),
        in_specs=[
            pl.BlockSpec((BN, HID), lambda i: (i, 0)),
            pl.BlockSpec((1, 1, BN), lambda i: (i, 0, 0)),
            pl.BlockSpec((HID, HID), lambda i: (0, 0)),
            pl.BlockSpec((1, HID), lambda i: (0, 0)),
            pl.BlockSpec((1, HID), lambda i: (0, 0)),
            pl.BlockSpec((1, 1), lambda i: (0, 0)),
        ],
        out_specs=pl.BlockSpec((NG, OUT), lambda i: (0, 0)),
        out_shape=jax.ShapeDtypeStruct((NG, OUT), F32),
        scratch_shapes=[pltpu.VMEM((NG, HID), F32)],
    )(h, bidx3, w1, b1, w2t, b2)


def kernel(x, pos, edge_attr, edge_index, batch_idx, emb_W, emb_b,
           msg_W1, msg_b1, msg_W2, msg_b2, upd_W1, upd_b1, upd_W2, upd_b2,
           pred_W1, pred_b1, pred_W2, pred_b2):
    src = edge_index[0]
    dst = edge_index[1]
    pos16 = jnp.concatenate([pos, jnp.zeros((N, 13), F32)], axis=1)
    zeros_n = jnp.zeros((NPAD, HID), F32)
    bidx3 = batch_idx.reshape(N // BN, 1, BN)

    h = _emb_call(x, emb_W, emb_b.reshape(1, HID))
    ps, pd = _sc_pos_gather(pos16, src, dst)

    for l in range(DEPTH):
        a, b = _proj_call(h, msg_W1[l, : 2 * HID])
        ga, gb = _sc_gather(a, b, src, dst)
        wrest = jnp.concatenate(
            [msg_W1[l, 2 * HID:], msg_b1[l].reshape(1, HID)], axis=0
        )
        m2 = _msg_call(ga, gb, ps, pd, edge_attr, wrest,
                       msg_W2[l], msg_b2[l].reshape(1, HID))
        partials = _sc_scatter(m2, dst, zeros_n)[:, :N, :]
        h = _upd_call(h, partials, upd_W1[l], upd_b1[l].reshape(1, HID),
                      upd_W2[l], upd_b2[l].reshape(1, HID))

    return _pool_call(h, bidx3, pred_W1, pred_b1.reshape(1, HID),
                      pred_W2.reshape(1, HID), pred_b2.reshape(1, 1))


# speedup vs baseline: 2.6491x; 1.0503x over previous
import functools

import jax
import jax.numpy as jnp
from jax import lax
from jax.experimental import pallas as pl
from jax.experimental.pallas import tpu as pltpu
from jax.experimental.pallas import tpu_sc as plsc

DEPTH = 4
HID = 128
NF = 128
NR = 4
OUT = 1
N = 10000
E = 320000
NG = 64

F32 = jnp.float32

NC = 2
NS = 16
NW = NC * NS
EPW = E // NW
CH = 80
NCHUNK = EPW // CH
NPAD = 10240
RPT = NPAD // NS

BN = 2000
BE = 2000

def _mesh():
    return plsc.VectorSubcoreMesh(
        core_axis_name="c", subcore_axis_name="s", num_cores=NC, num_subcores=NS
    )


def _silu(x):
    return x * lax.logistic(x)


def _ln(x):
    m = jnp.mean(x, axis=-1, keepdims=True)
    v = jnp.mean((x - m) * (x - m), axis=-1, keepdims=True)
    return (x - m) * lax.rsqrt(v + 1e-5)


_SC_CACHE = {}
RB = 5
CHS = 40
NCHUNKS = EPW // CHS


def _sc_gather_body(a_hbm, b_hbm, src_hbm, dst_hbm, ga_hbm, *scr):
    src_all, dst_all = scr[0], scr[1]
    abuf = scr[2:2 + RB]
    gsa = scr[2 + RB:2 + 2 * RB]
    gsb = scr[2 + 2 * RB:2 + 3 * RB]
    wsa = scr[2 + 3 * RB:2 + 4 * RB]
    c = lax.axis_index("c")
    s = lax.axis_index("s")
    base = (c * NS + s) * EPW
    pltpu.sync_copy(src_hbm.at[pl.ds(base, EPW)], src_all)
    pltpu.sync_copy(dst_hbm.at[pl.ds(base, EPW)], dst_all)

    def outer(g, carry):
        for k in range(RB):
            i = g * RB + k
            loc = i * CH
            off = base + loc

            @pl.when(g > 0)
            def _():
                pltpu.make_async_copy(abuf[k], ga_hbm.at[pl.ds(off, CH)], wsa[k]).wait()

            pltpu.async_copy(a_hbm.at[src_all.at[pl.ds(loc, CH)]], abuf[k], gsa[k])
        for k in range(RB):
            i = g * RB + k
            loc = i * CH
            off = base + loc
            pltpu.make_async_copy(a_hbm.at[src_all.at[pl.ds(loc, CH)]], abuf[k], gsa[k]).wait()
            pltpu.async_copy(b_hbm.at[dst_all.at[pl.ds(loc, CH)]], abuf[k], gsb[k], add=True)
        for k in range(RB):
            i = g * RB + k
            loc = i * CH
            off = base + loc
            pltpu.make_async_copy(b_hbm.at[dst_all.at[pl.ds(loc, CH)]], abuf[k], gsb[k]).wait()
            pltpu.async_copy(abuf[k], ga_hbm.at[pl.ds(off, CH)], wsa[k])
        return carry

    lax.fori_loop(0, NCHUNK // RB, outer, 0)
    for k in range(RB):
        pltpu.make_async_copy(abuf[k], ga_hbm.at[pl.ds(base, CH)], wsa[k]).wait()


def _sc_gather(a, b, src, dst):
    k = _SC_CACHE.get("gather")
    if k is None:
        scr = (
            [pltpu.VMEM((EPW,), jnp.int32)] * 2
            + [pltpu.VMEM((CH, HID), F32)] * RB
            + [pltpu.SemaphoreType.DMA] * (3 * RB)
        )
        k = pl.kernel(
            _sc_gather_body,
            out_type=jax.ShapeDtypeStruct((E, HID), F32),
            mesh=_mesh(),
            scratch_types=scr,
        )
        _SC_CACHE["gather"] = k
    return k(a, b, src, dst)


def _sc_pos_gather_body(p_hbm, src_hbm, dst_hbm, ps_hbm, pd_hbm, *scr):
    src_all, dst_all = scr[0], scr[1]
    abuf = scr[2:2 + RB]
    bbuf = scr[2 + RB:2 + 2 * RB]
    gsa = scr[2 + 2 * RB:2 + 3 * RB]
    gsb = scr[2 + 3 * RB:2 + 4 * RB]
    wsa = scr[2 + 4 * RB:2 + 5 * RB]
    wsb = scr[2 + 5 * RB:2 + 6 * RB]
    c = lax.axis_index("c")
    s = lax.axis_index("s")
    base = (c * NS + s) * EPW
    pltpu.sync_copy(src_hbm.at[pl.ds(base, EPW)], src_all)
    pltpu.sync_copy(dst_hbm.at[pl.ds(base, EPW)], dst_all)

    def outer(g, carry):
        for k in range(RB):
            i = g * RB + k
            loc = i * CH
            off = base + loc

            @pl.when(g > 0)
            def _():
                pltpu.make_async_copy(abuf[k], ps_hbm.at[pl.ds(off, CH)], wsa[k]).wait()
                pltpu.make_async_copy(bbuf[k], pd_hbm.at[pl.ds(off, CH)], wsb[k]).wait()

            pltpu.async_copy(p_hbm.at[src_all.at[pl.ds(loc, CH)]], abuf[k], gsa[k])
            pltpu.async_copy(p_hbm.at[dst_all.at[pl.ds(loc, CH)]], bbuf[k], gsb[k])
        for k in range(RB):
            i = g * RB + k
            loc = i * CH
            off = base + loc
            pltpu.make_async_copy(p_hbm.at[src_all.at[pl.ds(loc, CH)]], abuf[k], gsa[k]).wait()
            pltpu.make_async_copy(p_hbm.at[dst_all.at[pl.ds(loc, CH)]], bbuf[k], gsb[k]).wait()
            pltpu.async_copy(abuf[k], ps_hbm.at[pl.ds(off, CH)], wsa[k])
            pltpu.async_copy(bbuf[k], pd_hbm.at[pl.ds(off, CH)], wsb[k])
        return carry

    lax.fori_loop(0, NCHUNK // RB, outer, 0)
    for k in range(RB):
        pltpu.make_async_copy(abuf[k], ps_hbm.at[pl.ds(base, CH)], wsa[k]).wait()
        pltpu.make_async_copy(bbuf[k], pd_hbm.at[pl.ds(base, CH)], wsb[k]).wait()


def _sc_pos_gather(pos16, src, dst):
    k = _SC_CACHE.get("pos")
    if k is None:
        scr = (
            [pltpu.VMEM((EPW,), jnp.int32)] * 2
            + [pltpu.VMEM((CH, 16), F32)] * (2 * RB)
            + [pltpu.SemaphoreType.DMA] * (4 * RB)
        )
        k = pl.kernel(
            _sc_pos_gather_body,
            out_type=(
                jax.ShapeDtypeStruct((E, 16), F32),
                jax.ShapeDtypeStruct((E, 16), F32),
            ),
            mesh=_mesh(),
            scratch_types=scr,
            compiler_params=pltpu.CompilerParams(use_tc_tiling_on_sc=False),
        )
        _SC_CACHE["pos"] = k
    return k(pos16, src, dst)


def _sc_scatter_body(m2_hbm, dst_hbm, zeros_hbm, out_hbm, *scr):
    acc_sh = scr[0]
    dbuf = scr[1:1 + RB]
    mbuf = scr[1 + RB:1 + 2 * RB]
    lsi = scr[1 + 2 * RB:1 + 3 * RB]
    lsm = scr[1 + 3 * RB:1 + 4 * RB]
    ssem = scr[1 + 4 * RB:1 + 5 * RB]
    c = lax.axis_index("c")
    s = lax.axis_index("s")
    base = (c * NS + s) * EPW

    pltpu.sync_copy(zeros_hbm.at[pl.ds(s * RPT, RPT)],
                    acc_sh.at[pl.ds(s * RPT, RPT)])
    plsc.subcore_barrier()

    def outer(g, carry):
        for k in range(RB):
            i = g * RB + k
            off = base + i * CHS

            @pl.when(g > 0)
            def _():
                pltpu.make_async_copy(mbuf[k], acc_sh.at[dbuf[k]], ssem[k]).wait()

            pltpu.async_copy(dst_hbm.at[pl.ds(off, CHS)], dbuf[k], lsi[k])
            pltpu.async_copy(m2_hbm.at[pl.ds(off, CHS)], mbuf[k], lsm[k])
        for k in range(RB):
            i = g * RB + k
            off = base + i * CHS
            pltpu.make_async_copy(dst_hbm.at[pl.ds(off, CHS)], dbuf[k], lsi[k]).wait()
            pltpu.make_async_copy(m2_hbm.at[pl.ds(off, CHS)], mbuf[k], lsm[k]).wait()
            pltpu.async_copy(mbuf[k], acc_sh.at[dbuf[k]], ssem[k], add=True)
        return carry

    lax.fori_loop(0, NCHUNKS // RB, outer, 0)
    for k in range(RB):
        pltpu.make_async_copy(mbuf[k], acc_sh.at[dbuf[k]], ssem[k]).wait()
    plsc.subcore_barrier()
    pltpu.sync_copy(acc_sh.at[pl.ds(s * RPT, RPT)],
                    out_hbm.at[c, pl.ds(s * RPT, RPT)])


def _sc_scatter(m2, dst, zeros_n):
    k = _SC_CACHE.get("scatter")
    if k is None:
        scr = (
            [pltpu.VMEM_SHARED((NPAD, HID), F32)]
            + [pltpu.VMEM((CHS,), jnp.int32)] * RB
            + [pltpu.VMEM((CHS, HID), F32)] * RB
            + [pltpu.SemaphoreType.DMA] * (3 * RB)
        )
        k = pl.kernel(
            _sc_scatter_body,
            out_type=jax.ShapeDtypeStruct((NC, NPAD, HID), F32),
            mesh=_mesh(),
            scratch_types=scr,
        )
        _SC_CACHE["scatter"] = k
    return k(m2, dst, zeros_n)


def _emb_body(x_ref, w_ref, b_ref, o_ref):
    o_ref[...] = (
        jnp.dot(x_ref[...], w_ref[...], preferred_element_type=F32, precision=lax.Precision.HIGHEST) + b_ref[...]
    )


def _emb_call(x, w, b):
    return pl.pallas_call(
        _emb_body,
        grid=(N // BN,),
        in_specs=[
            pl.BlockSpec((BN, NF), lambda i: (i, 0)),
            pl.BlockSpec((NF, HID), lambda i: (0, 0)),
            pl.BlockSpec((1, HID), lambda i: (0, 0)),
        ],
        out_specs=pl.BlockSpec((BN, HID), lambda i: (i, 0)),
        out_shape=jax.ShapeDtypeStruct((N, HID), F32),
    )(x, w, b)


def _proj_body(h_ref, w_ref, a_ref, b_ref):
    h = h_ref[...]
    w = w_ref[...]
    a_ref[...] = jnp.dot(h, w[:HID], preferred_element_type=F32, precision=lax.Precision.HIGHEST)
    b_ref[...] = jnp.dot(h, w[HID:2 * HID], preferred_element_type=F32, precision=lax.Precision.HIGHEST)


def _proj_call(h, w1):
    return pl.pallas_call(
        _proj_body,
        grid=(N // BN,),
        in_specs=[
            pl.BlockSpec((BN, HID), lambda i: (i, 0)),
            pl.BlockSpec((2 * HID, HID), lambda i: (0, 0)),
        ],
        out_specs=[
            pl.BlockSpec((BN, HID), lambda i: (i, 0)),
            pl.BlockSpec((BN, HID), lambda i: (i, 0)),
        ],
        out_shape=[
            jax.ShapeDtypeStruct((N, HID), F32),
            jax.ShapeDtypeStruct((N, HID), F32),
        ],
    )(h, w1)


def _msg_body(ga_ref, ps_ref, pd_ref, at_ref, wr_ref, w2_ref, b2_ref,
              o_ref):
    d = ps_ref[...] - pd_ref[...]
    d2 = jnp.sum(d * d, axis=-1, keepdims=True)
    a = at_ref[...]
    mx = jnp.max(a, axis=-1, keepdims=True)
    eq = a >= mx
    e0 = eq[:, 0:1]
    e1 = eq[:, 1:2] & ~e0
    e2 = eq[:, 2:3] & ~(e0 | e1)
    e3 = eq[:, 3:4] & ~(e0 | e1 | e2)
    wr = wr_ref[...]
    pre = (
        ga_ref[...]
        + d2 * wr[0:1]
        + e0.astype(F32) * wr[1:2]
        + e1.astype(F32) * wr[2:3]
        + e2.astype(F32) * wr[3:4]
        + e3.astype(F32) * wr[4:5]
        + wr[5:6]
    )
    m = _silu(_ln(pre))
    y = jnp.dot(m, w2_ref[...], preferred_element_type=F32, precision=lax.Precision.HIGHEST) + b2_ref[...]
    o_ref[...] = _silu(y)


def _msg_call(ga, ps, pd, attr, wrest, w2, b2):
    return pl.pallas_call(
        _msg_body,
        grid=(E // BE,),
        in_specs=[
            pl.BlockSpec((BE, HID), lambda i: (i, 0)),
            pl.BlockSpec((BE, 16), lambda i: (i, 0)),
            pl.BlockSpec((BE, 16), lambda i: (i, 0)),
            pl.BlockSpec((BE, NR), lambda i: (i, 0)),
            pl.BlockSpec((6, HID), lambda i: (0, 0)),
            pl.BlockSpec((HID, HID), lambda i: (0, 0)),
            pl.BlockSpec((1, HID), lambda i: (0, 0)),
        ],
        out_specs=pl.BlockSpec((BE, HID), lambda i: (i, 0)),
        out_shape=jax.ShapeDtypeStruct((E, HID), F32),
    )(ga, ps, pd, attr, wrest, w2, b2)


def _upd_body(h_ref, p_ref, u1_ref, ub1_ref, u2_ref, ub2_ref, o_ref):
    h = h_ref[...]
    agg = p_ref[0] + p_ref[1]
    u1 = u1_ref[...]
    pre = (
        jnp.dot(h, u1[:HID], preferred_element_type=F32, precision=lax.Precision.HIGHEST)
        + jnp.dot(agg, u1[HID:], preferred_element_type=F32, precision=lax.Precision.HIGHEST)
        + ub1_ref[...]
    )
    u = _silu(_ln(pre))
    o_ref[...] = h + jnp.dot(u, u2_ref[...], preferred_element_type=F32, precision=lax.Precision.HIGHEST) + ub2_ref[...]


def _upd_call(h, partials, u1, ub1, u2, ub2):
    return pl.pallas_call(
        _upd_body,
        grid=(N // BN,),
        in_specs=[
            pl.BlockSpec((BN, HID), lambda i: (i, 0)),
            pl.BlockSpec((NC, BN, HID), lambda i: (0, i, 0)),
            pl.BlockSpec((2 * HID, HID), lambda i: (0, 0)),
            pl.BlockSpec((1, HID), lambda i: (0, 0)),
            pl.BlockSpec((HID, HID), lambda i: (0, 0)),
            pl.BlockSpec((1, HID), lambda i: (0, 0)),
        ],
        out_specs=pl.BlockSpec((BN, HID), lambda i: (i, 0)),
        out_shape=jax.ShapeDtypeStruct((N, HID), F32),
    )(h, partials, u1, ub1, u2, ub2)


def _pool_body(h_ref, bi_ref, w1_ref, b1_ref, w2t_ref, b2_ref, o_ref, acc):
    i = pl.program_id(0)

    @pl.when(i == 0)
    def _():
        acc[...] = jnp.zeros((NG, HID), F32)

    b = bi_ref[0]
    g = lax.broadcasted_iota(jnp.int32, (NG, BN), 0)
    oht = (g == b).astype(F32)
    acc[...] += jnp.dot(oht, h_ref[...], preferred_element_type=F32, precision=lax.Precision.HIGHEST)

    @pl.when(i == N // BN - 1)
    def _():
        z = jnp.dot(acc[...], w1_ref[...], preferred_element_type=F32, precision=lax.Precision.HIGHEST) + b1_ref[...]
        z = jnp.maximum(z, 0.0)
        o_ref[...] = jnp.sum(z * w2t_ref[...], axis=-1, keepdims=True) + b2_ref[...]


def _pool_call(h, bidx3, w1, b1, w2t, b2):
    return pl.pallas_call(
        _pool_body,
        grid=(N // BN,),
        in_specs=[
            pl.BlockSpec((BN, HID), lambda i: (i, 0)),
            pl.BlockSpec((1, 1, BN), lambda i: (i, 0, 0)),
            pl.BlockSpec((HID, HID), lambda i: (0, 0)),
            pl.BlockSpec((1, HID), lambda i: (0, 0)),
            pl.BlockSpec((1, HID), lambda i: (0, 0)),
            pl.BlockSpec((1, 1), lambda i: (0, 0)),
        ],
        out_specs=pl.BlockSpec((NG, OUT), lambda i: (0, 0)),
        out_shape=jax.ShapeDtypeStruct((NG, OUT), F32),
        scratch_shapes=[pltpu.VMEM((NG, HID), F32)],
    )(h, bidx3, w1, b1, w2t, b2)


def kernel(x, pos, edge_attr, edge_index, batch_idx, emb_W, emb_b,
           msg_W1, msg_b1, msg_W2, msg_b2, upd_W1, upd_b1, upd_W2, upd_b2,
           pred_W1, pred_b1, pred_W2, pred_b2):
    src = edge_index[0]
    dst = edge_index[1]
    pos16 = jnp.concatenate([pos, jnp.zeros((N, 13), F32)], axis=1)
    zeros_n = jnp.zeros((NPAD, HID), F32)
    bidx3 = batch_idx.reshape(N // BN, 1, BN)

    h = _emb_call(x, emb_W, emb_b.reshape(1, HID))
    ps, pd = _sc_pos_gather(pos16, src, dst)

    for l in range(DEPTH):
        a, b = _proj_call(h, msg_W1[l, : 2 * HID])
        ga = _sc_gather(a, b, src, dst)
        wrest = jnp.concatenate(
            [msg_W1[l, 2 * HID:], msg_b1[l].reshape(1, HID)], axis=0
        )
        m2 = _msg_call(ga, ps, pd, edge_attr, wrest,
                       msg_W2[l], msg_b2[l].reshape(1, HID))
        partials = _sc_scatter(m2, dst, zeros_n)[:, :N, :]
        h = _upd_call(h, partials, upd_W1[l], upd_b1[l].reshape(1, HID),
                      upd_W2[l], upd_b2[l].reshape(1, HID))

    return _pool_call(h, bidx3, pred_W1, pred_b1.reshape(1, HID),
                      pred_W2.reshape(1, HID), pred_b2.reshape(1, 1))
